# baseline (speedup 1.0000x reference)
import functools

import jax
import jax.numpy as jnp
from jax import lax
from jax.experimental import pallas as pl
from jax.experimental.pallas import tpu as pltpu
from jax.experimental.pallas import tpu_sc as plsc

_VOCAB = 100000
_EMBED = 128
_HIDDEN = 128
_BATCH = 1024
_SEQ = 200

_NC = 2
_NS = 16
_NW = _NC * _NS
_TOTAL = _SEQ * _BATCH
_BPW = _TOTAL // _NW
_CH = 128
_NCH = _BPW // _CH


def _sc_gather_body(table_hbm, idx_hbm, out_hbm, idx_v, buf0, buf1, gsem0, gsem1):
    wid = lax.axis_index("s") * _NC + lax.axis_index("c")
    base = wid * _BPW
    pltpu.sync_copy(idx_hbm.at[wid], idx_v)

    def pair(i, carry):
        j0 = 2 * i
        j1 = j0 + 1
        g0 = pltpu.async_copy(table_hbm.at[idx_v.at[j0]], buf0, gsem0)
        g1 = pltpu.async_copy(table_hbm.at[idx_v.at[j1]], buf1, gsem1)
        g0.wait()
        pltpu.sync_copy(buf0, out_hbm.at[pl.ds(base + j0 * _CH, _CH)])
        g1.wait()
        pltpu.sync_copy(buf1, out_hbm.at[pl.ds(base + j1 * _CH, _CH)])
        return carry

    lax.fori_loop(0, _NCH // 2, pair, 0)


def _sc_gather(table, idx2):
    mesh = plsc.VectorSubcoreMesh(core_axis_name="c", subcore_axis_name="s")
    run = functools.partial(
        pl.kernel,
        mesh=mesh,
        out_type=jax.ShapeDtypeStruct((_TOTAL, _EMBED), jnp.float32),
        scratch_types=[
            pltpu.VMEM((_NCH, _CH), jnp.int32),
            pltpu.VMEM((_CH, _EMBED), jnp.float32),
            pltpu.VMEM((_CH, _EMBED), jnp.float32),
            pltpu.SemaphoreType.DMA,
            pltpu.SemaphoreType.DMA,
        ],
    )(_sc_gather_body)
    return run(table, idx2)


_TB = 8


def _gru_body(x_ref, wih_ref, whh_ref, bsum_ref, bihn_ref, out_ref, h_ref):
    t = pl.program_id(0)

    @pl.when(t == 0)
    def _init():
        h_ref[...] = jnp.zeros_like(h_ref)

    h = h_ref[...]
    wih = wih_ref[...]
    whh = whh_ref[...]
    bsum = bsum_ref[...]
    bihn = bihn_ref[...]
    H = _HIDDEN
    for k in range(_TB):
        gi = jnp.dot(x_ref[k].astype(jnp.bfloat16), wih,
                     preferred_element_type=jnp.float32)
        gh = jnp.dot(h.astype(jnp.bfloat16), whh,
                     preferred_element_type=jnp.float32) + bsum
        r = 0.5 * jnp.tanh(0.5 * (gi[:, 0:H] + gh[:, 0:H])) + 0.5
        z = 0.5 * jnp.tanh(0.5 * (gi[:, H:2 * H] + gh[:, H:2 * H])) + 0.5
        n = jnp.tanh(gi[:, 2 * H:] + (r * gh[:, 2 * H:] + bihn))
        h = n + z * (h - n)
    h_ref[...] = h

    @pl.when(t == pl.num_programs(0) - 1)
    def _emit():
        out_ref[...] = h


def _tc_gru(xs, W_ih, W_hh, b_ih, b_hh):
    return pl.pallas_call(
        _gru_body,
        grid=(_SEQ // _TB,),
        in_specs=[
            pl.BlockSpec((_TB, _BATCH, _EMBED), lambda t: (t, 0, 0)),
            pl.BlockSpec((_EMBED, 3 * _HIDDEN), lambda t: (0, 0)),
            pl.BlockSpec((_HIDDEN, 3 * _HIDDEN), lambda t: (0, 0)),
            pl.BlockSpec((1, 3 * _HIDDEN), lambda t: (0, 0)),
            pl.BlockSpec((1, _HIDDEN), lambda t: (0, 0)),
        ],
        out_specs=pl.BlockSpec((_BATCH, _HIDDEN), lambda t: (0, 0)),
        out_shape=jax.ShapeDtypeStruct((_BATCH, _HIDDEN), jnp.float32),
        scratch_shapes=[pltpu.VMEM((_BATCH, _HIDDEN), jnp.float32)],
    )(xs, W_ih.astype(jnp.bfloat16), W_hh.astype(jnp.bfloat16),
      jnp.concatenate([b_ih[:2 * _HIDDEN] + b_hh[:2 * _HIDDEN],
                       b_hh[2 * _HIDDEN:]]).reshape(1, -1),
      b_ih[2 * _HIDDEN:].reshape(1, -1))


def kernel(source, table, W_ih, W_hh, b_ih, b_hh):
    idx = source.astype(jnp.int32).T.reshape(_TOTAL)
    idx2 = idx.reshape(_NW, _NCH, _CH)
    gathered = _sc_gather(table, idx2)
    xs = gathered.reshape(_SEQ, _BATCH, _EMBED)
    return _tc_gru(xs, W_ih, W_hh, b_ih, b_hh)


# speedup vs baseline: 1.2607x; 1.2607x over previous
import functools

import jax
import jax.numpy as jnp
from jax import lax
from jax.experimental import pallas as pl
from jax.experimental.pallas import tpu as pltpu
from jax.experimental.pallas import tpu_sc as plsc

_VOCAB = 100000
_EMBED = 128
_HIDDEN = 128
_BATCH = 1024
_SEQ = 200

_NC = 2
_NS = 16
_NW = _NC * _NS

_NCHUNKS = 5
_SEQ_C = _SEQ // _NCHUNKS
_CTOTAL = _SEQ_C * _BATCH
_BPW = _CTOTAL // _NW
_CH = 128
_NCH = _BPW // _CH


def _sc_gather_body(table_hbm, idx_hbm, out_hbm, idx_v, buf0, buf1,
                    gsem0, gsem1, osem0, osem1):
    wid = lax.axis_index("s") * _NC + lax.axis_index("c")
    base = wid * _BPW
    pltpu.sync_copy(idx_hbm.at[wid], idx_v)

    def pair(i, carry):
        j0 = 2 * i
        j1 = j0 + 1

        @pl.when(i > 0)
        def _drain():
            pltpu.make_async_copy(
                buf0, out_hbm.at[pl.ds(base + (j0 - 2) * _CH, _CH)],
                osem0).wait()
            pltpu.make_async_copy(
                buf1, out_hbm.at[pl.ds(base + (j1 - 2) * _CH, _CH)],
                osem1).wait()

        g0 = pltpu.async_copy(table_hbm.at[idx_v.at[j0]], buf0, gsem0)
        g1 = pltpu.async_copy(table_hbm.at[idx_v.at[j1]], buf1, gsem1)
        g0.wait()
        pltpu.async_copy(buf0, out_hbm.at[pl.ds(base + j0 * _CH, _CH)], osem0)
        g1.wait()
        pltpu.async_copy(buf1, out_hbm.at[pl.ds(base + j1 * _CH, _CH)], osem1)
        return carry

    lax.fori_loop(0, _NCH // 2, pair, 0)
    last0 = (_NCH - 2) * _CH
    last1 = (_NCH - 1) * _CH
    pltpu.make_async_copy(
        buf0, out_hbm.at[pl.ds(base + last0, _CH)], osem0).wait()
    pltpu.make_async_copy(
        buf1, out_hbm.at[pl.ds(base + last1, _CH)], osem1).wait()


def _sc_gather(table, idx2):
    mesh = plsc.VectorSubcoreMesh(core_axis_name="c", subcore_axis_name="s")
    run = functools.partial(
        pl.kernel,
        mesh=mesh,
        out_type=jax.ShapeDtypeStruct((_CTOTAL, _EMBED), jnp.float32),
        scratch_types=[
            pltpu.VMEM((_NCH, _CH), jnp.int32),
            pltpu.VMEM((_CH, _EMBED), jnp.float32),
            pltpu.VMEM((_CH, _EMBED), jnp.float32),
            pltpu.SemaphoreType.DMA,
            pltpu.SemaphoreType.DMA,
            pltpu.SemaphoreType.DMA,
            pltpu.SemaphoreType.DMA,
        ],
    )(_sc_gather_body)
    return run(table, idx2)


_TB = 8


def _gru_body(x_ref, h0_ref, wih_ref, whh_ref, bsum_ref, bihn_ref,
              out_ref, h_ref):
    t = pl.program_id(0)

    @pl.when(t == 0)
    def _init():
        h_ref[...] = h0_ref[...]

    h = h_ref[...]
    wih = wih_ref[...]
    whh = whh_ref[...]
    bsum = bsum_ref[...]
    bihn = bihn_ref[...]
    H = _HIDDEN
    for k in range(_TB):
        gi = jnp.dot(x_ref[k].astype(jnp.bfloat16), wih,
                     preferred_element_type=jnp.float32)
        gh = jnp.dot(h.astype(jnp.bfloat16), whh,
                     preferred_element_type=jnp.float32) + bsum
        r = 0.5 * jnp.tanh(0.5 * (gi[:, 0:H] + gh[:, 0:H])) + 0.5
        z = 0.5 * jnp.tanh(0.5 * (gi[:, H:2 * H] + gh[:, H:2 * H])) + 0.5
        n = jnp.tanh(gi[:, 2 * H:] + (r * gh[:, 2 * H:] + bihn))
        h = n + z * (h - n)
    h_ref[...] = h

    @pl.when(t == pl.num_programs(0) - 1)
    def _emit():
        out_ref[...] = h


def _tc_gru(xs, h0, wih_bf, whh_bf, bsum, bihn):
    return pl.pallas_call(
        _gru_body,
        grid=(_SEQ_C // _TB,),
        in_specs=[
            pl.BlockSpec((_TB, _BATCH, _EMBED), lambda t: (t, 0, 0)),
            pl.BlockSpec((_BATCH, _HIDDEN), lambda t: (0, 0)),
            pl.BlockSpec((_EMBED, 3 * _HIDDEN), lambda t: (0, 0)),
            pl.BlockSpec((_HIDDEN, 3 * _HIDDEN), lambda t: (0, 0)),
            pl.BlockSpec((1, 3 * _HIDDEN), lambda t: (0, 0)),
            pl.BlockSpec((1, _HIDDEN), lambda t: (0, 0)),
        ],
        out_specs=pl.BlockSpec((_BATCH, _HIDDEN), lambda t: (0, 0)),
        out_shape=jax.ShapeDtypeStruct((_BATCH, _HIDDEN), jnp.float32),
        scratch_shapes=[pltpu.VMEM((_BATCH, _HIDDEN), jnp.float32)],
    )(xs, h0, wih_bf, whh_bf, bsum, bihn)


def kernel(source, table, W_ih, W_hh, b_ih, b_hh):
    idx = source.astype(jnp.int32).T.reshape(_NCHUNKS, _NW, _NCH, _CH)
    wih_bf = W_ih.astype(jnp.bfloat16)
    whh_bf = W_hh.astype(jnp.bfloat16)
    bsum = jnp.concatenate([b_ih[:2 * _HIDDEN] + b_hh[:2 * _HIDDEN],
                            b_hh[2 * _HIDDEN:]]).reshape(1, -1)
    bihn = b_ih[2 * _HIDDEN:].reshape(1, -1)

    h = jnp.zeros((_BATCH, _HIDDEN), jnp.float32)
    gathered = [_sc_gather(table, idx[c]) for c in range(_NCHUNKS)]
    for c in range(_NCHUNKS):
        xs = gathered[c].reshape(_SEQ_C, _BATCH, _EMBED)
        h = _tc_gru(xs, h, wih_bf, whh_bf, bsum, bihn)
    return h
